# baseline (speedup 1.0000x reference)
import functools

import jax
import jax.numpy as jnp
from jax import lax
from jax.experimental import pallas as pl
from jax.experimental.pallas import tpu as pltpu
from jax.experimental.pallas import tpu_sc as plsc

_NPTS = 320000
_NSEG = 32768
_CIN = 9
_CHID = 64

_NW = 32
_SEGB = 512
_NHALF = _NSEG // _SEGB
_CHUNK = 256
_FBLK = 3200
_FIN_BLK = 2048


def _gelu_exact(x):
    return 0.5 * x * (1.0 + lax.erf(x * 0.7071067811865476))


_FROWS = _FBLK // 2


def _ffn_body(xe_ref, xo_ref, idx_ref, w1l_ref, w1r_ref, w2_ref, mavg_ref,
              bb1_ref, bb2_ref, g2_ref, be2_ref, h_ref, bnd_ref):
    h = (jnp.dot(xe_ref[...], w1l_ref[...], preferred_element_type=jnp.float32)
         + jnp.dot(xo_ref[...], w1r_ref[...], preferred_element_type=jnp.float32)
         + bb1_ref[...])
    h = _gelu_exact(h)
    h = jnp.dot(h, w2_ref[...], preferred_element_type=jnp.float32) + bb2_ref[...]
    h = _gelu_exact(h)
    mu = jnp.dot(h, mavg_ref[...], preferred_element_type=jnp.float32)
    ex2 = jnp.dot(h * h, mavg_ref[...], preferred_element_type=jnp.float32)
    var = ex2 - mu * mu
    h_ref[...] = (h - mu) * lax.rsqrt(var + 1e-5) * g2_ref[...] + be2_ref[...]
    row = idx_ref[pl.ds(pl.program_id(0), 1), :]
    thr = lax.broadcasted_iota(jnp.int32, (128, 1), 0) * _SEGB
    part = jnp.sum((row < thr).astype(jnp.int32), axis=1,
                   keepdims=True)
    part8 = jnp.broadcast_to(part.reshape(1, 128), (8, 128))

    @pl.when(pl.program_id(0) == 0)
    def _():
        bnd_ref[...] = jnp.zeros((8, 128), jnp.int32)

    bnd_ref[...] = bnd_ref[...] + part8


def _ffn(pc, idx, W1, b1, W2, b2, g, b, interpret=False):
    grid = _NPTS // _FBLK
    half = _NPTS // 2
    pc2 = jnp.concatenate([pc[0::2], pc[1::2]], axis=0)
    zc = jnp.zeros_like(W1)
    w1l = jnp.concatenate([W1, zc], axis=1)
    w1r = jnp.concatenate([zc, W1], axis=1)
    zw = jnp.zeros_like(W2)
    w2b = jnp.concatenate([jnp.concatenate([W2, zw], axis=1),
                           jnp.concatenate([zw, W2], axis=1)], axis=0)
    ones64 = jnp.full((_CHID, _CHID), 1.0 / _CHID, jnp.float32)
    z64 = jnp.zeros((_CHID, _CHID), jnp.float32)
    mavg = jnp.concatenate([jnp.concatenate([ones64, z64], axis=1),
                            jnp.concatenate([z64, ones64], axis=1)], axis=0)
    b1b = jnp.tile(b1, 2).reshape(1, 128)
    b2b = jnp.tile(b2, 2).reshape(1, 128)
    g2 = jnp.tile(g, 2).reshape(1, 128)
    be2 = jnp.tile(b, 2).reshape(1, 128)
    nblk = _FROWS
    h, bnd = pl.pallas_call(
        _ffn_body,
        grid=(grid,),
        in_specs=[
            pl.BlockSpec((nblk, _CIN), lambda i: (i, 0)),
            pl.BlockSpec((nblk, _CIN), lambda i, _g=grid: (_g + i, 0)),
            pl.BlockSpec((grid, _FBLK), lambda i: (0, 0)),
            pl.BlockSpec((_CIN, 128), lambda i: (0, 0)),
            pl.BlockSpec((_CIN, 128), lambda i: (0, 0)),
            pl.BlockSpec((128, 128), lambda i: (0, 0)),
            pl.BlockSpec((128, 128), lambda i: (0, 0)),
            pl.BlockSpec((1, 128), lambda i: (0, 0)),
            pl.BlockSpec((1, 128), lambda i: (0, 0)),
            pl.BlockSpec((1, 128), lambda i: (0, 0)),
            pl.BlockSpec((1, 128), lambda i: (0, 0)),
        ],
        out_specs=[
            pl.BlockSpec((nblk, 128), lambda i: (i, 0)),
            pl.BlockSpec((8, 128), lambda i: (0, 0)),
        ],
        out_shape=[
            jax.ShapeDtypeStruct((half, 128), jnp.float32),
            jax.ShapeDtypeStruct((8, 128), jnp.int32),
        ],
        interpret=interpret,
    )(pc2, pc2, idx.reshape(grid, _FBLK), w1l, w1r, w2b, mavg, b1b, b2b, g2,
      be2)
    return h, bnd


def _sc_body(h_hbm, idx_hbm, bnd_hbm, sums, sqs, maxs, cnts,
             hbuf, ibuf, rstart, rseg, ptmp, asum, asq, amax, acnt, bnd):
    _NG = _CHID // 16
    wid = lax.axis_index("s") * 2 + lax.axis_index("c")
    pltpu.sync_copy(bnd_hbm, bnd)
    zero = jnp.zeros((16,), jnp.float32)
    lane = lax.iota(jnp.int32, 16)
    ptmp[pl.ds(0, 16)] = jnp.zeros((16,), jnp.int32)

    for half in range(2):
        hb = wid * 2 + half
        seg_base = hb * _SEGB
        bv = bnd[pl.ds(hb, 16)]
        s = bv[0]
        e = bv[1]

        def zacc(r, _):
            o = pl.ds(16 * r, 16)
            asum[o] = zero
            asq[o] = zero
            amax[o] = zero
            return 0

        lax.fori_loop(0, _SEGB * _CHID // 16, zacc, 0, unroll=8)

        def zcnt(r, _):
            acnt[pl.ds(16 * r, 16)] = zero
            return 0

        lax.fori_loop(0, _SEGB, zcnt, 0, unroll=8)

        k0 = lax.div(s, _CHUNK)
        k1 = lax.div(e + _CHUNK - 1, _CHUNK)

        def chunk_body(k, _):
            p = k * _CHUNK
            pltpu.sync_copy(h_hbm.at[pl.ds(p * _CHID, _CHUNK * _CHID)], hbuf)
            ibuf[pl.ds(0, 16)] = jnp.full((16,), -1, jnp.int32)
            pltpu.sync_copy(idx_hbm.at[pl.ds(p, _CHUNK)], ibuf.at[pl.ds(8, _CHUNK)])
            lo = jnp.maximum(s - p, 0)
            hi = jnp.minimum(e - p, _CHUNK)

            g0 = lax.div(lo, 16)
            g1 = lax.div(hi + 15, 16)

            one16 = jnp.zeros((16,), jnp.int32) + 1
            zero16 = jnp.zeros((16,), jnp.int32)
            trash16 = jnp.zeros((16,), jnp.int32) + (_CHUNK + 24)

            def ph1(j, wp):
                iv = ibuf[pl.ds(j * 16 + 8, 16)]
                pv = ibuf[pl.ds(j * 16 + 7, 16)]
                pos = j * 16 + lane
                bmask = ((iv != pv) | (pos == lo)) & (pos >= lo) & (pos < hi)
                ind = jnp.where(bmask, one16, zero16)
                o = ind
                for sh in (1, 2, 4, 8):
                    ptmp[pl.ds(8, 16)] = o
                    o = o + ptmp[pl.ds(8 - sh, 16)]
                sl = iv - seg_base
                for l in range(16):
                    il = ind[l]

                    @pl.when(il == 1)
                    def _(l=l):
                        d = wp + o[l] - 1
                        rstart[pl.ds(d, 16)] = zero16 + (j * 16 + l)
                        rseg[pl.ds(d, 16)] = zero16 + sl[l]

                return wp + o[15]

            wp = lax.fori_loop(g0, g1, ph1, jnp.int32(0))
            rstart[pl.ds(wp, 16)] = jnp.zeros((16,), jnp.int32) + hi

            def run_body(r, _):
                a = rstart[pl.ds(r, 16)][0]
                b = rstart[pl.ds(r + 1, 16)][0]
                sg = rseg[pl.ds(r, 16)][0]

                def pt(i, acc):
                    vs = [hbuf[pl.ds(i * _CHID + 16 * g, 16)]
                          for g in range(_NG)]
                    sp = [acc[g] + vs[g] for g in range(_NG)]
                    qp = [acc[_NG + g] + vs[g] * vs[g] for g in range(_NG)]
                    mp = [jnp.maximum(acc[2 * _NG + g], vs[g])
                          for g in range(_NG)]
                    return sp + qp + mp

                acc = lax.fori_loop(a, b, pt, [zero] * (3 * _NG))
                co = pl.ds(sg * 16, 16)
                acnt[co] = acnt[co] + (b - a).astype(jnp.float32)
                for g in range(_NG):
                    c = pl.ds(sg * _CHID + 16 * g, 16)
                    asum[c] = asum[c] + acc[g]
                    asq[c] = asq[c] + acc[_NG + g]
                    amax[c] = jnp.maximum(amax[c], acc[2 * _NG + g])
                return 0

            lax.fori_loop(0, wp, run_body, 0)
            return 0

        lax.fori_loop(k0, k1, chunk_body, 0)

        pltpu.sync_copy(asum, sums.at[pl.ds(seg_base * _CHID, _SEGB * _CHID)])
        pltpu.sync_copy(asq, sqs.at[pl.ds(seg_base * _CHID, _SEGB * _CHID)])
        pltpu.sync_copy(amax, maxs.at[pl.ds(seg_base * _CHID, _SEGB * _CHID)])
        pltpu.sync_copy(acnt, cnts.at[pl.ds(seg_base * 16, _SEGB * 16)])


def _sc_reduce(h, idx, bounds, interpret=False):
    mesh = plsc.VectorSubcoreMesh(core_axis_name="c", subcore_axis_name="s")
    kern = pl.kernel(
        _sc_body,
        mesh=mesh,
        out_type=[
            jax.ShapeDtypeStruct((_NSEG * _CHID,), jnp.float32),
            jax.ShapeDtypeStruct((_NSEG * _CHID,), jnp.float32),
            jax.ShapeDtypeStruct((_NSEG * _CHID,), jnp.float32),
            jax.ShapeDtypeStruct((_NSEG * 16,), jnp.float32),
        ],
        scratch_types=[
            pltpu.VMEM((_CHUNK * _CHID,), jnp.float32),
            pltpu.VMEM((_CHUNK + 24,), jnp.int32),
            pltpu.VMEM((_CHUNK + 32,), jnp.int32),
            pltpu.VMEM((_CHUNK + 32,), jnp.int32),
            pltpu.VMEM((32,), jnp.int32),
            pltpu.VMEM((_SEGB * _CHID,), jnp.float32),
            pltpu.VMEM((_SEGB * _CHID,), jnp.float32),
            pltpu.VMEM((_SEGB * _CHID,), jnp.float32),
            pltpu.VMEM((_SEGB * 16,), jnp.float32),
            pltpu.VMEM((80,), jnp.int32),
        ],
        interpret=interpret,
    )
    return kern(h, idx, bounds)


def _fin_body(sums_ref, sqs_ref, maxs_ref, cnts_ref, out_ref):
    cnt = cnts_ref[:, 0:1]
    denom = jnp.where(cnt > 0.0, cnt, 1.0)
    mean = sums_ref[...] / denom
    var = sqs_ref[...] / denom - mean * mean
    std = jnp.where(var > 0.0,
                    jnp.sqrt(jnp.where(var > 0.0, var, 1.0)),
                    0.0)
    fmax = jnp.maximum(maxs_ref[...], 0.0)
    out_ref[...] = jnp.concatenate([fmax, mean, std], axis=-1)


def _finalize(sums, sqs, maxs, cnts, interpret=False):
    grid = _NSEG // _FIN_BLK
    return pl.pallas_call(
        _fin_body,
        grid=(grid,),
        in_specs=[
            pl.BlockSpec((_FIN_BLK, _CHID), lambda i: (i, 0)),
            pl.BlockSpec((_FIN_BLK, _CHID), lambda i: (i, 0)),
            pl.BlockSpec((_FIN_BLK, _CHID), lambda i: (i, 0)),
            pl.BlockSpec((_FIN_BLK, 16), lambda i: (i, 0)),
        ],
        out_specs=pl.BlockSpec((_FIN_BLK, 3 * _CHID), lambda i: (i, 0)),
        out_shape=jax.ShapeDtypeStruct((_NSEG, 3 * _CHID), jnp.float32),
        interpret=interpret,
    )(sums, sqs, maxs, cnts)


def kernel(pc, i_frustum, W1, b1, W2, b2, ln_gamma, ln_beta):
    idx = i_frustum.astype(jnp.int32)
    h, bnd_hist = _ffn(pc, idx, W1, b1, W2, b2, ln_gamma, ln_beta)
    bounds = bnd_hist[0, :80]
    sums, sqs, maxs, cnts = _sc_reduce(h.reshape(-1), idx, bounds)
    return _finalize(sums.reshape(_NSEG, _CHID), sqs.reshape(_NSEG, _CHID),
                     maxs.reshape(_NSEG, _CHID), cnts.reshape(_NSEG, 16))


# speedup vs baseline: 1.7148x; 1.7148x over previous
import functools

import jax
import jax.numpy as jnp
from jax import lax
from jax.experimental import pallas as pl
from jax.experimental.pallas import tpu as pltpu
from jax.experimental.pallas import tpu_sc as plsc

_NPTS = 320000
_NSEG = 32768
_CIN = 9
_CHID = 64

_NW = 32
_SEGB = 512
_NHALF = _NSEG // _SEGB
_CHUNK = 256
_FBLK = 3200
_FIN_BLK = 2048


def _gelu_exact(x):
    return 0.5 * x * (1.0 + lax.erf(x * 0.7071067811865476))


def _ffn_body(pc_ref, idx_ref, w1_ref, b1_ref, w2_ref, b2_ref, g_ref, bb_ref,
              h_ref, bnd_ref):
    x = pc_ref[...]
    h = jnp.dot(x, w1_ref[...], preferred_element_type=jnp.float32) + b1_ref[...]
    h = _gelu_exact(h)
    h = jnp.dot(h, w2_ref[...], preferred_element_type=jnp.float32) + b2_ref[...]
    h = _gelu_exact(h)
    mu = jnp.mean(h, axis=-1, keepdims=True)
    var = jnp.mean((h - mu) ** 2, axis=-1, keepdims=True)
    h_ref[...] = (h - mu) * lax.rsqrt(var + 1e-5) * g_ref[...] + bb_ref[...]
    row = idx_ref[pl.ds(pl.program_id(0), 1), :]
    thr = lax.broadcasted_iota(jnp.int32, (128, 1), 0) * _SEGB
    part = jnp.sum((row < thr).astype(jnp.int32), axis=1,
                   keepdims=True)
    part8 = jnp.broadcast_to(part.reshape(1, 128), (8, 128))

    @pl.when(pl.program_id(0) == 0)
    def _():
        bnd_ref[...] = jnp.zeros((8, 128), jnp.int32)

    bnd_ref[...] = bnd_ref[...] + part8


def _ffn(pc, idx, W1, b1, W2, b2, g, b, interpret=False):
    grid = _NPTS // _FBLK
    return pl.pallas_call(
        _ffn_body,
        grid=(grid,),
        in_specs=[
            pl.BlockSpec((_FBLK, _CIN), lambda i: (i, 0)),
            pl.BlockSpec((grid, _FBLK), lambda i: (0, 0)),
            pl.BlockSpec((_CIN, _CHID), lambda i: (0, 0)),
            pl.BlockSpec((1, _CHID), lambda i: (0, 0)),
            pl.BlockSpec((_CHID, _CHID), lambda i: (0, 0)),
            pl.BlockSpec((1, _CHID), lambda i: (0, 0)),
            pl.BlockSpec((1, _CHID), lambda i: (0, 0)),
            pl.BlockSpec((1, _CHID), lambda i: (0, 0)),
        ],
        out_specs=[
            pl.BlockSpec((_FBLK, _CHID), lambda i: (i, 0)),
            pl.BlockSpec((8, 128), lambda i: (0, 0)),
        ],
        out_shape=[
            jax.ShapeDtypeStruct((_NPTS, _CHID), jnp.float32),
            jax.ShapeDtypeStruct((8, 128), jnp.int32),
        ],
        interpret=interpret,
    )(pc, idx.reshape(grid, _FBLK), W1, b1.reshape(1, -1), W2,
      b2.reshape(1, -1), g.reshape(1, -1), b.reshape(1, -1))


def _sc_body(h_hbm, idx_hbm, bnd_hbm, sums, sqs, maxs, cnts,
             hbuf, ibuf, rstart, rseg, ptmp, asum, asq, amax, acnt, bnd):
    _NG = _CHID // 16
    wid = lax.axis_index("s") * 2 + lax.axis_index("c")
    pltpu.sync_copy(bnd_hbm, bnd)
    zero = jnp.zeros((16,), jnp.float32)
    lane = lax.iota(jnp.int32, 16)
    ptmp[pl.ds(0, 16)] = jnp.zeros((16,), jnp.int32)

    for half in range(2):
        hb = wid * 2 + half
        seg_base = hb * _SEGB
        bv = bnd[pl.ds(hb, 16)]
        s = bv[0]
        e = bv[1]

        def zacc(r, _):
            o = pl.ds(16 * r, 16)
            asum[o] = zero
            asq[o] = zero
            amax[o] = zero
            return 0

        lax.fori_loop(0, _SEGB * _CHID // 16, zacc, 0, unroll=8)

        def zcnt(r, _):
            acnt[pl.ds(16 * r, 16)] = zero
            return 0

        lax.fori_loop(0, _SEGB, zcnt, 0, unroll=8)

        k0 = lax.div(s, _CHUNK)
        k1 = lax.div(e + _CHUNK - 1, _CHUNK)

        def chunk_body(k, _):
            p = k * _CHUNK
            pltpu.sync_copy(h_hbm.at[pl.ds(p * _CHID, _CHUNK * _CHID)], hbuf)
            ibuf[pl.ds(0, 16)] = jnp.full((16,), -1, jnp.int32)
            pltpu.sync_copy(idx_hbm.at[pl.ds(p, _CHUNK)], ibuf.at[pl.ds(8, _CHUNK)])
            lo = jnp.maximum(s - p, 0)
            hi = jnp.minimum(e - p, _CHUNK)

            g0 = lax.div(lo, 16)
            g1 = lax.div(hi + 15, 16)

            one16 = jnp.zeros((16,), jnp.int32) + 1
            zero16 = jnp.zeros((16,), jnp.int32)
            trash16 = jnp.zeros((16,), jnp.int32) + (_CHUNK + 24)

            def ph1(j, wp):
                iv = ibuf[pl.ds(j * 16 + 8, 16)]
                pv = ibuf[pl.ds(j * 16 + 7, 16)]
                pos = j * 16 + lane
                bmask = ((iv != pv) | (pos == lo)) & (pos >= lo) & (pos < hi)
                ind = jnp.where(bmask, one16, zero16)
                o = ind
                for sh in (1, 2, 4, 8):
                    ptmp[pl.ds(8, 16)] = o
                    o = o + ptmp[pl.ds(8 - sh, 16)]
                sl = iv - seg_base
                for l in range(16):
                    il = ind[l]

                    @pl.when(il == 1)
                    def _(l=l):
                        d = wp + o[l] - 1
                        rstart[pl.ds(d, 16)] = zero16 + (j * 16 + l)
                        rseg[pl.ds(d, 16)] = zero16 + sl[l]

                return wp + o[15]

            wp = lax.fori_loop(g0, g1, ph1, jnp.int32(0))
            rstart[pl.ds(wp, 16)] = jnp.zeros((16,), jnp.int32) + hi

            def run_body(r, _):
                a = rstart[pl.ds(r, 16)][0]
                b = rstart[pl.ds(r + 1, 16)][0]
                sg = rseg[pl.ds(r, 16)][0]

                def pt(i, acc):
                    vs = [hbuf[pl.ds(i * _CHID + 16 * g, 16)]
                          for g in range(_NG)]
                    sp = [acc[g] + vs[g] for g in range(_NG)]
                    qp = [acc[_NG + g] + vs[g] * vs[g] for g in range(_NG)]
                    mp = [jnp.maximum(acc[2 * _NG + g], vs[g])
                          for g in range(_NG)]
                    return sp + qp + mp

                acc = lax.fori_loop(a, b, pt, [zero] * (3 * _NG))
                co = pl.ds(sg * 16, 16)
                acnt[co] = acnt[co] + (b - a).astype(jnp.float32)
                for g in range(_NG):
                    c = pl.ds(sg * _CHID + 16 * g, 16)
                    asum[c] = asum[c] + acc[g]
                    asq[c] = asq[c] + acc[_NG + g]
                    amax[c] = jnp.maximum(amax[c], acc[2 * _NG + g])
                return 0

            lax.fori_loop(0, wp, run_body, 0)
            return 0

        lax.fori_loop(k0, k1, chunk_body, 0)

        pltpu.sync_copy(asum, sums.at[pl.ds(seg_base * _CHID, _SEGB * _CHID)])
        pltpu.sync_copy(asq, sqs.at[pl.ds(seg_base * _CHID, _SEGB * _CHID)])
        pltpu.sync_copy(amax, maxs.at[pl.ds(seg_base * _CHID, _SEGB * _CHID)])
        pltpu.sync_copy(acnt, cnts.at[pl.ds(seg_base * 16, _SEGB * 16)])


def _sc_reduce(h, idx, bounds, interpret=False):
    mesh = plsc.VectorSubcoreMesh(core_axis_name="c", subcore_axis_name="s")
    kern = pl.kernel(
        _sc_body,
        mesh=mesh,
        out_type=[
            jax.ShapeDtypeStruct((_NSEG * _CHID,), jnp.float32),
            jax.ShapeDtypeStruct((_NSEG * _CHID,), jnp.float32),
            jax.ShapeDtypeStruct((_NSEG * _CHID,), jnp.float32),
            jax.ShapeDtypeStruct((_NSEG * 16,), jnp.float32),
        ],
        scratch_types=[
            pltpu.VMEM((_CHUNK * _CHID,), jnp.float32),
            pltpu.VMEM((_CHUNK + 24,), jnp.int32),
            pltpu.VMEM((_CHUNK + 32,), jnp.int32),
            pltpu.VMEM((_CHUNK + 32,), jnp.int32),
            pltpu.VMEM((32,), jnp.int32),
            pltpu.VMEM((_SEGB * _CHID,), jnp.float32),
            pltpu.VMEM((_SEGB * _CHID,), jnp.float32),
            pltpu.VMEM((_SEGB * _CHID,), jnp.float32),
            pltpu.VMEM((_SEGB * 16,), jnp.float32),
            pltpu.VMEM((80,), jnp.int32),
        ],
        interpret=interpret,
    )
    return kern(h, idx, bounds)


def _fin_body(sums_ref, sqs_ref, maxs_ref, cnts_ref, out_ref):
    cnt = cnts_ref[:, 0:1]
    denom = jnp.where(cnt > 0.0, cnt, 1.0)
    mean = sums_ref[...] / denom
    var = sqs_ref[...] / denom - mean * mean
    std = jnp.where(var > 0.0,
                    jnp.sqrt(jnp.where(var > 0.0, var, 1.0)),
                    0.0)
    fmax = jnp.maximum(maxs_ref[...], 0.0)
    out_ref[...] = jnp.concatenate([fmax, mean, std], axis=-1)


def _finalize(sums, sqs, maxs, cnts, interpret=False):
    grid = _NSEG // _FIN_BLK
    return pl.pallas_call(
        _fin_body,
        grid=(grid,),
        in_specs=[
            pl.BlockSpec((_FIN_BLK, _CHID), lambda i: (i, 0)),
            pl.BlockSpec((_FIN_BLK, _CHID), lambda i: (i, 0)),
            pl.BlockSpec((_FIN_BLK, _CHID), lambda i: (i, 0)),
            pl.BlockSpec((_FIN_BLK, 16), lambda i: (i, 0)),
        ],
        out_specs=pl.BlockSpec((_FIN_BLK, 3 * _CHID), lambda i: (i, 0)),
        out_shape=jax.ShapeDtypeStruct((_NSEG, 3 * _CHID), jnp.float32),
        interpret=interpret,
    )(sums, sqs, maxs, cnts)


def kernel(pc, i_frustum, W1, b1, W2, b2, ln_gamma, ln_beta):
    idx = i_frustum.astype(jnp.int32)
    h, bnd_hist = _ffn(pc, idx, W1, b1, W2, b2, ln_gamma, ln_beta)
    bounds = bnd_hist[0, :80]
    sums, sqs, maxs, cnts = _sc_reduce(h.reshape(-1), idx, bounds)
    return _finalize(sums.reshape(_NSEG, _CHID), sqs.reshape(_NSEG, _CHID),
                     maxs.reshape(_NSEG, _CHID), cnts.reshape(_NSEG, 16))
